# baseline (speedup 1.0000x reference)
import functools

import jax
import jax.numpy as jnp
from jax import lax
from jax.experimental import pallas as pl
from jax.experimental.pallas import tpu as pltpu
from jax.experimental.pallas import tpu_sc as plsc

_N = 10000
_D = 128
_E = 320000
_NC = 2
_NS = 16
_NW = _NC * _NS
_K = 128
_CH = -(-_E // (_NW * _K))
_EP = _CH * _K
_P = _EP * _NW
_NA = 10240
_ZR = _NA // _NS
_LAG = 4


def _sc_mesh():
    return plsc.VectorSubcoreMesh(
        core_axis_name="c", subcore_axis_name="s",
        num_cores=_NC, num_subcores=_NS)


@functools.cache
def _sc_degree_kernel():
    return pl.kernel(
        _sc_degree_body,
        out_type=jax.ShapeDtypeStruct((_NC, _NA), jnp.float32),
        mesh=_sc_mesh(),
        scratch_types=[
            pltpu.VMEM_SHARED((_NA,), jnp.float32),
            pltpu.VMEM((_CH, _K), jnp.int32),
            pltpu.VMEM((_K,), jnp.float32),
            pltpu.SemaphoreType.DMA,
        ],
    )


def _sc_degree_body(src_hbm, z_hbm, out_hbm, deg_sh, sidx, ones_v, sem):
    c = lax.axis_index("c")
    s = lax.axis_index("s")
    w = c * _NS + s
    for j in range(_K // 16):
        ones_v[pl.ds(j * 16, 16)] = jnp.ones((16,), jnp.float32)
    pltpu.sync_copy(z_hbm, deg_sh.at[pl.ds(s * _ZR, _ZR)])
    pltpu.sync_copy(src_hbm.at[w], sidx)
    plsc.subcore_barrier()

    @pl.loop(0, _CH)
    def _fire(i):
        @pl.when(i >= _LAG)
        def _lagged_drain():
            pltpu.make_async_copy(ones_v, deg_sh.at[sidx.at[0]], sem).wait()
        pltpu.async_copy(ones_v, deg_sh.at[sidx.at[i]], sem, add=True)

    @pl.loop(0, min(_LAG, _CH))
    def _drain(i):
        pltpu.make_async_copy(ones_v, deg_sh.at[sidx.at[0]], sem).wait()

    plsc.subcore_barrier()
    pltpu.sync_copy(deg_sh.at[pl.ds(s * _ZR, _ZR)], out_hbm.at[c, pl.ds(s * _ZR, _ZR)])


_EPT = _E // _NW
_CHF = _EPT // _K
_TL = _EPT - _CHF * _K
_NDUM = _NA - _N


@functools.cache
def _sc_aggregate_kernel():
    return pl.kernel(
        _sc_aggregate_body,
        out_type=jax.ShapeDtypeStruct((_NC, _NA, _D), jnp.float32),
        mesh=_sc_mesh(),
        scratch_types=[
            pltpu.VMEM_SHARED((_NA, _D), jnp.float32),
            pltpu.VMEM((_CH * _K,), jnp.int32),
            pltpu.VMEM((2, _K), jnp.int32),
            pltpu.VMEM((_K,), jnp.int32),
            pltpu.VMEM((2, _K, _D), jnp.float32),
            pltpu.SemaphoreType.DMA,
            pltpu.SemaphoreType.DMA,
        ],
    )


def _sc_aggregate_body(y_hbm, ei_hbm, z_hbm, out_hbm,
                       acc_sh, sidx, didx, didx_t, rows, gsem0, gsem1):
    c = lax.axis_index("c")
    s = lax.axis_index("s")
    w = c * _NS + s
    pltpu.sync_copy(ei_hbm.at[pl.ds(w * _EPT, _EPT)], sidx.at[pl.ds(0, _EPT)])
    lanes = lax.iota(jnp.int32, 16)
    for j in range((_K - _TL) // 16):
        sidx[pl.ds(_EPT + j * 16, 16)] = _TL + j * 16 + lanes
    for j in range(_K // 16):
        didx_t[pl.ds(j * 16, 16)] = _N + (j * 16 + lanes + s * 13) % _NDUM
    pltpu.sync_copy(z_hbm, acc_sh.at[pl.ds(s * _ZR, _ZR)])
    plsc.subcore_barrier()

    gsems = (gsem0, gsem1)

    def _start_gather(i, b):
        return pltpu.async_copy(
            y_hbm.at[sidx.at[pl.ds(i * _K, _K)]], rows.at[b], gsems[b])

    def _wait_gather(b):
        pltpu.make_async_copy(
            y_hbm.at[sidx.at[pl.ds(0, _K)]], rows.at[b], gsems[b]).wait()

    _start_gather(0, 0)

    @pl.loop(0, _CH, step=2)
    def _chunk(i):
        for b in range(2):
            nb = 1 - b
            @pl.when(i + b + 1 < _CH)
            def _prefetch():
                _start_gather(i + b + 1, nb)
            dbase = _E + w * _EPT
            @pl.when(i + b < _CHF)
            def _full():
                pltpu.sync_copy(ei_hbm.at[pl.ds(dbase + (i + b) * _K, _K)], didx.at[b])
                _wait_gather(b)
                pltpu.sync_copy(rows.at[b], acc_sh.at[didx.at[b]], add=True)
            @pl.when(i + b == _CHF)
            def _tail():
                pltpu.sync_copy(ei_hbm.at[pl.ds(dbase + _CHF * _K, _TL)],
                                didx_t.at[pl.ds(0, _TL)])
                _wait_gather(b)
                pltpu.sync_copy(rows.at[b], acc_sh.at[didx_t], add=True)

    plsc.subcore_barrier()
    pltpu.sync_copy(acc_sh.at[pl.ds(s * _ZR, _ZR)], out_hbm.at[c, pl.ds(s * _ZR, _ZR)])


def _tc_prescale_body(x_ref, d_ref, y_ref):
    d = d_ref[0] + d_ref[1] + 1.0
    y_ref[...] = x_ref[...] * lax.rsqrt(d)


def _tc_prescale(x, deg2c, rb=1000):
    return pl.pallas_call(
        _tc_prescale_body,
        grid=(_N // rb,),
        in_specs=[
            pl.BlockSpec((rb, _D), lambda i: (i, 0)),
            pl.BlockSpec((_NC, rb, 1), lambda i: (0, i, 0)),
        ],
        out_specs=pl.BlockSpec((rb, _D), lambda i: (i, 0)),
        out_shape=jax.ShapeDtypeStruct((_N, _D), jnp.float32),
    )(x, deg2c)


def _tc_combine_body(x_ref, d_ref, a_ref, o_ref):
    d = d_ref[0] + d_ref[1] + 1.0
    a = a_ref[0] + a_ref[1]
    xv = x_ref[...]
    o_ref[:, :_D] = xv
    o_ref[:, _D:] = a * lax.rsqrt(d) + xv / d


def _tc_combine(x, deg2c, acc2, rb=1000):
    return pl.pallas_call(
        _tc_combine_body,
        grid=(_N // rb,),
        in_specs=[
            pl.BlockSpec((rb, _D), lambda i: (i, 0)),
            pl.BlockSpec((_NC, rb, 1), lambda i: (0, i, 0)),
            pl.BlockSpec((_NC, rb, _D), lambda i: (0, i, 0)),
        ],
        out_specs=pl.BlockSpec((rb, 2 * _D), lambda i: (i, 0)),
        out_shape=jax.ShapeDtypeStruct((_N, 2 * _D), jnp.float32),
    )(x, deg2c, acc2)


def kernel(x, edge_index, features_idx):
    src = edge_index[0]
    pad = _P - _E
    dummy = _N + (jnp.arange(pad, dtype=jnp.int32) % _NDUM)
    src_deg = jnp.concatenate([src, dummy]).reshape(_NW, _CH, _K)
    em = edge_index.reshape(2 * _E)
    zeros1 = jnp.zeros((_ZR,), jnp.float32)
    zeros2 = jnp.zeros((_ZR, _D), jnp.float32)

    deg2 = _sc_degree_kernel()(src_deg, zeros1)
    deg2c = deg2.reshape(_NC, _NA, 1)
    y = _tc_prescale(x, deg2c)
    acc2 = _sc_aggregate_kernel()(y, em, zeros2)
    return _tc_combine(x, deg2c, acc2)


# speedup vs baseline: 1.1694x; 1.1694x over previous
import functools

import jax
import jax.numpy as jnp
from jax import lax
from jax.experimental import pallas as pl
from jax.experimental.pallas import tpu as pltpu
from jax.experimental.pallas import tpu_sc as plsc

_N = 10000
_D = 128
_E = 320000
_NC = 2
_NS = 16
_NW = _NC * _NS
_K = 128
_CH = -(-_E // (_NW * _K))
_EP = _CH * _K
_P = _EP * _NW
_NA = 10240
_ZR = _NA // _NS
_LAG = 4
_EPT = _E // _NW
_CHF = _EPT // _K
_TL = _EPT - _CHF * _K
_NDUM = _NA - _N


def _sc_mesh():
    return plsc.VectorSubcoreMesh(
        core_axis_name="c", subcore_axis_name="s",
        num_cores=_NC, num_subcores=_NS)


@functools.cache
def _sc_degree_kernel():
    return pl.kernel(
        _sc_degree_body,
        out_type=jax.ShapeDtypeStruct((_NC, _NA), jnp.float32),
        mesh=_sc_mesh(),
        scratch_types=[
            pltpu.VMEM_SHARED((_NA,), jnp.float32),
            pltpu.VMEM((_CHF * _K,), jnp.int32),
            pltpu.VMEM((_K,), jnp.int32),
            pltpu.VMEM((_K,), jnp.float32),
            pltpu.SemaphoreType.DMA,
        ],
    )


def _sc_degree_body(ei_hbm, z_hbm, out_hbm, deg_sh, sidx, sidx_t, ones_v, sem):
    c = lax.axis_index("c")
    s = lax.axis_index("s")
    w = c * _NS + s
    for j in range(_K // 16):
        ones_v[pl.ds(j * 16, 16)] = jnp.ones((16,), jnp.float32)
    lanes = lax.iota(jnp.int32, 16)
    for j in range(_K // 16):
        sidx_t[pl.ds(j * 16, 16)] = _N + (j * 16 + lanes + s * 13) % _NDUM
    pltpu.sync_copy(z_hbm, deg_sh.at[pl.ds(s * _ZR, _ZR)])
    pltpu.sync_copy(ei_hbm.at[pl.ds(w * _EPT, _CHF * _K)], sidx)
    pltpu.sync_copy(ei_hbm.at[pl.ds(w * _EPT + _CHF * _K, _TL)],
                    sidx_t.at[pl.ds(0, _TL)])
    plsc.subcore_barrier()

    @pl.loop(0, _CHF)
    def _fire(i):
        @pl.when(i >= _LAG)
        def _lagged_drain():
            pltpu.make_async_copy(ones_v, deg_sh.at[sidx.at[pl.ds(0, _K)]], sem).wait()
        pltpu.async_copy(ones_v, deg_sh.at[sidx.at[pl.ds(i * _K, _K)]], sem, add=True)

    pltpu.async_copy(ones_v, deg_sh.at[sidx_t], sem, add=True)

    @pl.loop(0, min(_LAG, _CHF) + 1)
    def _drain(i):
        pltpu.make_async_copy(ones_v, deg_sh.at[sidx.at[pl.ds(0, _K)]], sem).wait()

    plsc.subcore_barrier()
    pltpu.sync_copy(deg_sh.at[pl.ds(s * _ZR, _ZR)], out_hbm.at[c, pl.ds(s * _ZR, _ZR)])


@functools.cache
def _sc_aggregate_kernel():
    return pl.kernel(
        _sc_aggregate_body,
        out_type=jax.ShapeDtypeStruct((_NC, _NA, _D), jnp.float32),
        mesh=_sc_mesh(),
        scratch_types=[
            pltpu.VMEM_SHARED((_NA, _D), jnp.float32),
            pltpu.VMEM((_CH * _K,), jnp.int32),
            pltpu.VMEM((2, _K), jnp.int32),
            pltpu.VMEM((_K,), jnp.int32),
            pltpu.VMEM((2, _K, _D), jnp.float32),
            pltpu.SemaphoreType.DMA,
            pltpu.SemaphoreType.DMA,
        ],
    )


def _sc_aggregate_body(y_hbm, ei_hbm, z_hbm, out_hbm,
                       acc_sh, sidx, didx, didx_t, rows, gsem0, gsem1):
    c = lax.axis_index("c")
    s = lax.axis_index("s")
    w = c * _NS + s
    pltpu.sync_copy(ei_hbm.at[pl.ds(w * _EPT, _EPT)], sidx.at[pl.ds(0, _EPT)])
    lanes = lax.iota(jnp.int32, 16)
    for j in range((_K - _TL) // 16):
        sidx[pl.ds(_EPT + j * 16, 16)] = _TL + j * 16 + lanes
    for j in range(_K // 16):
        didx_t[pl.ds(j * 16, 16)] = _N + (j * 16 + lanes + s * 13) % _NDUM
    pltpu.sync_copy(z_hbm, acc_sh.at[pl.ds(s * _ZR, _ZR)])
    plsc.subcore_barrier()

    gsems = (gsem0, gsem1)

    def _start_gather(i, b):
        return pltpu.async_copy(
            y_hbm.at[sidx.at[pl.ds(i * _K, _K)]], rows.at[b], gsems[b])

    def _wait_gather(b):
        pltpu.make_async_copy(
            y_hbm.at[sidx.at[pl.ds(0, _K)]], rows.at[b], gsems[b]).wait()

    _start_gather(0, 0)

    @pl.loop(0, _CH, step=2)
    def _chunk(i):
        for b in range(2):
            nb = 1 - b
            @pl.when(i + b + 1 < _CH)
            def _prefetch():
                _start_gather(i + b + 1, nb)
            dbase = _E + w * _EPT
            @pl.when(i + b < _CHF)
            def _full():
                pltpu.sync_copy(ei_hbm.at[pl.ds(dbase + (i + b) * _K, _K)], didx.at[b])
                _wait_gather(b)
                pltpu.sync_copy(rows.at[b], acc_sh.at[didx.at[b]], add=True)
            @pl.when(i + b == _CHF)
            def _tail():
                pltpu.sync_copy(ei_hbm.at[pl.ds(dbase + _CHF * _K, _TL)],
                                didx_t.at[pl.ds(0, _TL)])
                _wait_gather(b)
                pltpu.sync_copy(rows.at[b], acc_sh.at[didx_t], add=True)

    plsc.subcore_barrier()
    pltpu.sync_copy(acc_sh.at[pl.ds(s * _ZR, _ZR)], out_hbm.at[c, pl.ds(s * _ZR, _ZR)])


def _d_broadcast(d_ref):
    dsum = d_ref[0:1, :] + d_ref[1:2, :] + 1.0
    db = lax.dot_general(dsum, jnp.ones((1, _D), jnp.float32),
                         (((0,), (0,)), ((), ())),
                         preferred_element_type=jnp.float32)
    return db[:_N, :]


def _tc_prescale_body(x_ref, d_ref, y_ref):
    db = _d_broadcast(d_ref)
    y_ref[...] = x_ref[...] * lax.rsqrt(db)


def _tc_prescale(x, deg2):
    return pl.pallas_call(
        _tc_prescale_body,
        out_shape=jax.ShapeDtypeStruct((_N, _D), jnp.float32),
    )(x, deg2)


def _tc_combine_body(x_ref, d_ref, a_ref, o_ref):
    db = _d_broadcast(d_ref)
    a = a_ref[0, :_N, :] + a_ref[1, :_N, :]
    xv = x_ref[...]
    o_ref[:, :_D] = xv
    o_ref[:, _D:] = a * lax.rsqrt(db) + xv / db


def _tc_combine(x, deg2, acc2):
    return pl.pallas_call(
        _tc_combine_body,
        out_shape=jax.ShapeDtypeStruct((_N, 2 * _D), jnp.float32),
    )(x, deg2, acc2)


def kernel(x, edge_index, features_idx):
    em = edge_index.reshape(2 * _E)
    zeros1 = jnp.zeros((_ZR,), jnp.float32)
    zeros2 = jnp.zeros((_ZR, _D), jnp.float32)

    deg2 = _sc_degree_kernel()(em, zeros1)
    y = _tc_prescale(x, deg2)
    acc2 = _sc_aggregate_kernel()(y, em, zeros2)
    return _tc_combine(x, deg2, acc2)
